# baseline (speedup 1.0000x reference)
import numpy as np

import jax
import jax.numpy as jnp
from jax import lax
from jax.experimental import pallas as pl
from jax.experimental.pallas import tpu as pltpu

ROWS = 64
COLS = 1_000_000
SUB = 8
LANES = COLS // SUB
W = 2048
NCHUNK = LANES // W
TAIL = LANES - NCHUNK * W
EPS = np.float32(1e-10)
BIG = np.int32(2**30)

_K0 = np.uint32(0)
_K1 = np.uint32(42)
_K2 = np.uint32(0x1BD11BDA ^ 42)
_ROT = (13, 15, 26, 6, 17, 29, 16, 24)


def _rotl(x, r):
    return lax.shift_left(x, np.uint32(r)) | lax.shift_right_logical(
        x, np.uint32(32 - r)
    )


def _threefry_bits(x1):
    x0 = jnp.zeros_like(x1)
    inject = ((_K1, _K2), (_K2, _K0), (_K0, _K1), (_K1, _K2), (_K2, _K0))
    for g in range(5):
        rots = _ROT[:4] if g % 2 == 0 else _ROT[4:]
        for r in rots:
            x0 = x0 + x1
            x1 = _rotl(x1, r)
            x1 = x1 ^ x0
        a, b = inject[g]
        x0 = x0 + a
        x1 = x1 + np.uint32(b + g + 1)
    return x0 ^ x1


def _scores(xc, seed0, t, vmax):
    bits = _threefry_bits(seed0)
    mant = lax.shift_right_logical(bits, np.uint32(9)) | np.uint32(0x3F800000)
    f = lax.bitcast_convert_type(mant, jnp.float32)
    neg_u = np.float32(1.0) - f
    den = EPS - jnp.log1p(neg_u)
    return (xc - vmax) - t * jnp.log(den)


def _row_kernel(temps_ref, logits_ref, out_ref, s_ref):
    r = pl.program_id(0)
    t_raw = temps_ref[r]
    t = jnp.where(t_raw == 0.0, np.float32(1.0), t_raw)
    row_base = r.astype(jnp.uint32) * np.uint32(COLS)

    col_all = lax.broadcasted_iota(jnp.int32, (SUB, LANES), 1)
    row_all = lax.broadcasted_iota(jnp.int32, (SUB, LANES), 0)
    flat_all = row_all * LANES + col_all

    x_all = logits_ref[...]
    vmax = jnp.max(x_all)
    greedy = jnp.min(jnp.where(x_all == vmax, flat_all, BIG))

    colw = lax.broadcasted_iota(jnp.uint32, (SUB, W), 1)
    roww = lax.broadcasted_iota(jnp.uint32, (SUB, W), 0) * np.uint32(LANES)
    w0u = roww + colw

    def body(j, carry):
        base = pl.multiple_of(j * W, W)
        xc = logits_ref[:, pl.ds(base, W)]
        seed0 = w0u + (base.astype(jnp.uint32) + row_base + np.uint32(42))
        s_ref[:, pl.ds(base, W)] = _scores(xc, seed0, t, vmax)
        return carry

    lax.fori_loop(0, NCHUNK, body, 0)

    colt = lax.broadcasted_iota(jnp.int32, (SUB, TAIL), 1)
    rowt = lax.broadcasted_iota(jnp.int32, (SUB, TAIL), 0) * LANES
    flat_t = rowt + NCHUNK * W + colt
    xt = logits_ref[:, pl.ds(NCHUNK * W, TAIL)]
    seed_t = flat_t.astype(jnp.uint32) + (row_base + np.uint32(42))
    s_ref[:, pl.ds(NCHUNK * W, TAIL)] = _scores(xt, seed_t, t, vmax)

    s_all = s_ref[...]
    smax = jnp.max(s_all)
    sample = jnp.min(jnp.where(s_all == smax, flat_all, BIG))

    out_ref[0, 0, 0] = jnp.where(t_raw == 0.0, greedy, sample)


@jax.jit
def kernel(logits, temperatures):
    x = logits.reshape(ROWS * SUB, LANES)
    out = pl.pallas_call(
        _row_kernel,
        grid=(ROWS,),
        in_specs=[
            pl.BlockSpec(memory_space=pltpu.SMEM),
            pl.BlockSpec((SUB, LANES), lambda r: (r, 0)),
        ],
        out_specs=pl.BlockSpec(
            (1, 1, 1), lambda r: (r, 0, 0), memory_space=pltpu.SMEM
        ),
        out_shape=jax.ShapeDtypeStruct((ROWS, 1, 1), jnp.int32),
        scratch_shapes=[pltpu.VMEM((SUB, LANES), jnp.float32)],
        compiler_params=pltpu.CompilerParams(
            dimension_semantics=("parallel",),
        ),
    )(temperatures, x)
    return out.reshape(ROWS)


# speedup vs baseline: 1.1613x; 1.1613x over previous
import numpy as np

import jax
import jax.numpy as jnp
from jax import lax
from jax.experimental import pallas as pl
from jax.experimental.pallas import tpu as pltpu

ROWS = 64
COLS = 1_000_000
SUB = 8
LANES = COLS // SUB
W = 2048
NCHUNK = LANES // W
TAIL = LANES - NCHUNK * W
EPS = np.float32(1e-10)
BIG = np.int32(2**30)
NEG_INF = np.float32("-inf")

_K0 = np.uint32(0)
_K1 = np.uint32(42)
_K2 = np.uint32(0x1BD11BDA ^ 42)
_ROT = (13, 15, 26, 6, 17, 29, 16, 24)


def _rotl(x, r):
    return lax.shift_left(x, np.uint32(r)) | lax.shift_right_logical(
        x, np.uint32(32 - r)
    )


def _threefry_bits(x1):
    x0 = jnp.zeros_like(x1)
    inject = ((_K1, _K2), (_K2, _K0), (_K0, _K1), (_K1, _K2), (_K2, _K0))
    for g in range(5):
        rots = _ROT[:4] if g % 2 == 0 else _ROT[4:]
        for r in rots:
            x0 = x0 + x1
            x1 = _rotl(x1, r)
            x1 = x1 ^ x0
        a, b = inject[g]
        x0 = x0 + a
        x1 = x1 + np.uint32(b + g + 1)
    return x0 ^ x1


def _scores(xc, seed0, t, vmax):
    bits = _threefry_bits(seed0)
    mant = lax.shift_right_logical(bits, np.uint32(9)) | np.uint32(0x3F800000)
    f = lax.bitcast_convert_type(mant, jnp.float32)
    neg_u = np.float32(1.0) - f
    den = EPS - jnp.log1p(neg_u)
    return (xc - vmax) - t * jnp.log(den)


def _row_kernel(temps_ref, logits_ref, out_ref):
    r = pl.program_id(0)
    t_raw = temps_ref[r]
    t = jnp.where(t_raw == 0.0, np.float32(1.0), t_raw)
    row_base = r.astype(jnp.uint32) * np.uint32(COLS)

    colw = lax.broadcasted_iota(jnp.int32, (SUB, W), 1)
    roww = lax.broadcasted_iota(jnp.int32, (SUB, W), 0) * LANES
    w0 = roww + colw
    w0u = w0.astype(jnp.uint32)
    colt = lax.broadcasted_iota(jnp.int32, (SUB, TAIL), 1)
    rowt = lax.broadcasted_iota(jnp.int32, (SUB, TAIL), 0) * LANES
    flat_t = rowt + NCHUNK * W + colt

    gm = jnp.full((SUB, W), NEG_INF, jnp.float32)
    gj = jnp.zeros((SUB, W), jnp.int32)
    for j in range(NCHUNK):
        xc = logits_ref[:, pl.ds(j * W, W)]
        upd = xc > gm
        gm = jnp.where(upd, xc, gm)
        gj = jnp.where(upd, np.int32(j), gj)
    xt = logits_ref[:, pl.ds(NCHUNK * W, TAIL)]
    vmax = jnp.maximum(jnp.max(gm), jnp.max(xt))
    g_main = jnp.min(jnp.where(gm == vmax, w0 + gj * W, BIG))
    g_tail = jnp.min(jnp.where(xt == vmax, flat_t, BIG))
    greedy = jnp.minimum(g_main, g_tail)

    m_s = jnp.full((SUB, W), NEG_INF, jnp.float32)
    js = jnp.zeros((SUB, W), jnp.int32)
    for j in range(NCHUNK):
        xc = logits_ref[:, pl.ds(j * W, W)]
        seed0 = w0u + np.uint32(j * W + 42) + row_base
        s = _scores(xc, seed0, t, vmax)
        upd = s > m_s
        m_s = jnp.where(upd, s, m_s)
        js = jnp.where(upd, np.int32(j), js)
    smax_main = jnp.max(m_s)
    idx_main = jnp.min(jnp.where(m_s == smax_main, w0 + js * W, BIG))

    seed_t = flat_t.astype(jnp.uint32) + (row_base + np.uint32(42))
    s_t = _scores(xt, seed_t, t, vmax)
    smax_t = jnp.max(s_t)
    idx_t = jnp.min(jnp.where(s_t == smax_t, flat_t, BIG))

    sample = jnp.where(
        smax_t > smax_main,
        idx_t,
        jnp.where(smax_t == smax_main, jnp.minimum(idx_t, idx_main), idx_main),
    )
    out_ref[0, 0, 0] = jnp.where(t_raw == 0.0, greedy, sample)


@jax.jit
def kernel(logits, temperatures):
    x = logits.reshape(ROWS * SUB, LANES)
    out = pl.pallas_call(
        _row_kernel,
        grid=(ROWS,),
        in_specs=[
            pl.BlockSpec(memory_space=pltpu.SMEM),
            pl.BlockSpec((SUB, LANES), lambda r: (r, 0)),
        ],
        out_specs=pl.BlockSpec(
            (1, 1, 1), lambda r: (r, 0, 0), memory_space=pltpu.SMEM
        ),
        out_shape=jax.ShapeDtypeStruct((ROWS, 1, 1), jnp.int32),
        compiler_params=pltpu.CompilerParams(
            dimension_semantics=("parallel",),
        ),
    )(temperatures, x)
    return out.reshape(ROWS)


# speedup vs baseline: 1.2065x; 1.0389x over previous
import numpy as np

import jax
import jax.numpy as jnp
from jax import lax
from jax.experimental import pallas as pl
from jax.experimental.pallas import tpu as pltpu

ROWS = 64
COLS = 1_000_000
SUB = 8
LANES = COLS // SUB
W = 2048
NCHUNK = LANES // W
TAIL = LANES - NCHUNK * W
EPS = np.float32(1e-10)
BIG = np.int32(2**30)
NEG_INF = np.float32("-inf")

_K0 = np.uint32(0)
_K1 = np.uint32(42)
_K2 = np.uint32(0x1BD11BDA ^ 42)
_ROT = (13, 15, 26, 6, 17, 29, 16, 24)


def _rotl(x, r):
    return lax.shift_left(x, np.uint32(r)) | lax.shift_right_logical(
        x, np.uint32(32 - r)
    )


def _threefry_bits(x1):
    x0 = jnp.zeros_like(x1)
    inject = ((_K1, _K2), (_K2, _K0), (_K0, _K1), (_K1, _K2), (_K2, _K0))
    for g in range(5):
        rots = _ROT[:4] if g % 2 == 0 else _ROT[4:]
        for r in rots:
            x0 = x0 + x1
            x1 = _rotl(x1, r)
            x1 = x1 ^ x0
        a, b = inject[g]
        x0 = x0 + a
        x1 = x1 + np.uint32(b + g + 1)
    return x0 ^ x1


def _scores(xc, seed0, t, vmax):
    bits = _threefry_bits(seed0)
    mant = lax.shift_right_logical(bits, np.uint32(9)) | np.uint32(0x3F800000)
    f = lax.bitcast_convert_type(mant, jnp.float32)
    neg_u = np.float32(1.0) - f
    den = EPS - jnp.log1p(neg_u)
    return (xc - vmax) - t * jnp.log(den)


def _row_kernel(temps_ref, logits_ref, out_ref):
    r = pl.program_id(0)
    t_raw = temps_ref[r]
    t = t_raw
    row_base = r.astype(jnp.uint32) * np.uint32(COLS)

    colw = lax.broadcasted_iota(jnp.int32, (SUB, W), 1)
    roww = lax.broadcasted_iota(jnp.int32, (SUB, W), 0) * LANES
    w0 = roww + colw
    w0u = w0.astype(jnp.uint32)
    colt = lax.broadcasted_iota(jnp.int32, (SUB, TAIL), 1)
    rowt = lax.broadcasted_iota(jnp.int32, (SUB, TAIL), 0) * LANES
    flat_t = rowt + NCHUNK * W + colt

    gm = jnp.full((SUB, W), NEG_INF, jnp.float32)
    for j in range(NCHUNK):
        gm = jnp.maximum(gm, logits_ref[:, pl.ds(j * W, W)])
    xt = logits_ref[:, pl.ds(NCHUNK * W, TAIL)]
    vmax = jnp.maximum(jnp.max(gm), jnp.max(xt))

    m_s = jnp.full((SUB, W), NEG_INF, jnp.float32)
    js = jnp.zeros((SUB, W), jnp.int32)
    for j in range(NCHUNK):
        xc = logits_ref[:, pl.ds(j * W, W)]
        seed0 = w0u + np.uint32(j * W + 42) + row_base
        s = _scores(xc, seed0, t, vmax)
        upd = s > m_s
        m_s = jnp.where(upd, s, m_s)
        js = jnp.where(upd, np.int32(j), js)
    smax_main = jnp.max(m_s)
    idx_main = jnp.min(jnp.where(m_s == smax_main, w0 + js * W, BIG))

    seed_t = flat_t.astype(jnp.uint32) + (row_base + np.uint32(42))
    s_t = _scores(xt, seed_t, t, vmax)
    smax_t = jnp.max(s_t)
    idx_t = jnp.min(jnp.where(s_t == smax_t, flat_t, BIG))

    sample = jnp.where(
        smax_t > smax_main,
        idx_t,
        jnp.where(smax_t == smax_main, jnp.minimum(idx_t, idx_main), idx_main),
    )
    out_ref[0, 0, 0] = sample


@jax.jit
def kernel(logits, temperatures):
    x = logits.reshape(ROWS * SUB, LANES)
    out = pl.pallas_call(
        _row_kernel,
        grid=(ROWS,),
        in_specs=[
            pl.BlockSpec(memory_space=pltpu.SMEM),
            pl.BlockSpec((SUB, LANES), lambda r: (r, 0)),
        ],
        out_specs=pl.BlockSpec(
            (1, 1, 1), lambda r: (r, 0, 0), memory_space=pltpu.SMEM
        ),
        out_shape=jax.ShapeDtypeStruct((ROWS, 1, 1), jnp.int32),
        compiler_params=pltpu.CompilerParams(
            dimension_semantics=("parallel",),
        ),
    )(temperatures, x)
    return out.reshape(ROWS)
